# baseline (speedup 1.0000x reference)
import functools

import jax
import jax.numpy as jnp
from jax import lax
from jax.experimental import pallas as pl
from jax.experimental.pallas import tpu as pltpu
from jax.experimental.pallas import tpu_sc as plsc

H = 1000
HP = 1024
D = 128
NFB = HP // D
N = 10000
NP = 10240
E = 320000
ECHUNK = 48
NTILES = 16
NCHUNK = 424
EPT = NCHUNK * ECHUNK
EP = EPT * NTILES
SLAB_N = 10112
STRIPE = SLAB_N // NTILES
BN = 256
BN0 = 1024


def _mm0_body(xb, wb, ob):
    ob[...] = jnp.dot(xb[...], wb[...], preferred_element_type=jnp.float32)[None]


def _mm0(xp, w0r):
    return pl.pallas_call(
        _mm0_body,
        grid=(NP // BN0, NFB),
        in_specs=[
            pl.BlockSpec((BN0, D), lambda i, j: (i, 0)),
            pl.BlockSpec((D, D), lambda i, j: (0, j)),
        ],
        out_specs=pl.BlockSpec((1, BN0, D), lambda i, j: (j, i, 0)),
        out_shape=jax.ShapeDtypeStruct((NFB, NP, D), jnp.float32),
    )(xp, w0r)


def _gru_math(gi, gh, hv):
    r = jax.nn.sigmoid(gi[:, :HP] + gh[:, :HP])
    z = jax.nn.sigmoid(gi[:, HP:2 * HP] + gh[:, HP:2 * HP])
    n = jnp.tanh(gi[:, 2 * HP:] + r * gh[:, 2 * HP:])
    return (1.0 - z) * n + z * hv


def _gates(aggb, hb, wih, whh, bi, bh):
    hv = hb[...]
    gh = jnp.dot(hv, whh[...], preferred_element_type=jnp.float32) + bh[...][0:1, :]
    wihv = wih[...]
    gi = bi[...][0:1, :]
    for k in range(NFB):
        gi = gi + jnp.dot(aggb[k], wihv[k * D:(k + 1) * D, :],
                          preferred_element_type=jnp.float32)
    return _gru_math(gi, gh, hv), hv


def _gru_next_body(aggb, hb, wih, whh, bi, bh, wn, hob, mob):
    hn, _ = _gates(aggb, hb, wih, whh, bi, bh)
    hob[...] = hn
    wnv = wn[...]
    for k in range(NFB):
        mob[k] = jnp.dot(hn, wnv[:, k * D:(k + 1) * D],
                         preferred_element_type=jnp.float32)


def _gru_next(agg3, h, wih, whh, bi, bh, wn):
    return pl.pallas_call(
        _gru_next_body,
        grid=(NP // BN,),
        in_specs=[
            pl.BlockSpec((NFB, BN, D), lambda i: (0, i, 0)),
            pl.BlockSpec((BN, HP), lambda i: (i, 0)),
            pl.BlockSpec((HP, 3 * HP), lambda i: (0, 0)),
            pl.BlockSpec((HP, 3 * HP), lambda i: (0, 0)),
            pl.BlockSpec((8, 3 * HP), lambda i: (0, 0)),
            pl.BlockSpec((8, 3 * HP), lambda i: (0, 0)),
            pl.BlockSpec((HP, HP), lambda i: (0, 0)),
        ],
        out_specs=[
            pl.BlockSpec((BN, HP), lambda i: (i, 0)),
            pl.BlockSpec((NFB, BN, D), lambda i: (0, i, 0)),
        ],
        out_shape=[
            jax.ShapeDtypeStruct((NP, HP), jnp.float32),
            jax.ShapeDtypeStruct((NFB, NP, D), jnp.float32),
        ],
    )(agg3, h, wih, whh, bi, bh, wn)


def _gru_final_body(aggb, hb, wih, whh, bi, bh, lp, ob):
    hn, _ = _gates(aggb, hb, wih, whh, bi, bh)
    ob[...] = jnp.dot(jnp.maximum(hn, 0.0), lp[...],
                      preferred_element_type=jnp.float32)


def _gru_final(agg3, h, wih, whh, bi, bh, lp):
    return pl.pallas_call(
        _gru_final_body,
        grid=(NP // BN,),
        in_specs=[
            pl.BlockSpec((NFB, BN, D), lambda i: (0, i, 0)),
            pl.BlockSpec((BN, HP), lambda i: (i, 0)),
            pl.BlockSpec((HP, 3 * HP), lambda i: (0, 0)),
            pl.BlockSpec((HP, 3 * HP), lambda i: (0, 0)),
            pl.BlockSpec((8, 3 * HP), lambda i: (0, 0)),
            pl.BlockSpec((8, 3 * HP), lambda i: (0, 0)),
            pl.BlockSpec((HP, D), lambda i: (0, 0)),
        ],
        out_specs=pl.BlockSpec((BN, D), lambda i: (i, 0)),
        out_shape=jax.ShapeDtypeStruct((NP, D), jnp.float32),
    )(agg3, h, wih, whh, bi, bh, lp)


@functools.cache
def _sc_agg_fn():
    mesh = plsc.VectorSubcoreMesh(core_axis_name="c", subcore_axis_name="s")

    NSET = 8

    @functools.partial(
        pl.kernel,
        mesh=mesh,
        out_type=jax.ShapeDtypeStruct((NFB * NP, D), jnp.float32),
        scratch_types=[
            pltpu.VMEM_SHARED((SLAB_N, D), jnp.float32),
        ] + [pltpu.VMEM((2, ECHUNK), jnp.int32) for _ in range(NSET)]
          + [pltpu.VMEM((ECHUNK,), jnp.float32) for _ in range(NSET)]
          + [pltpu.VMEM((ECHUNK, D), jnp.float32) for _ in range(2)]
          + [pltpu.VMEM((ECHUNK, D), jnp.float32) for _ in range(2)]
          + [pltpu.SemaphoreType.DMA for _ in range(4 + NSET)],
    )
    def _sc_agg(m_hbm, sd_hbm, attr_hbm, zeros_hbm, agg_hbm,
                slab,
                e0, e1, e2, e3, e4, e5, e6, e7,
                a0, a1, a2, a3, a4, a5, a6, a7,
                gb0, gb1, sb0, sb1,
                g0, g1, s0, s1, t0, t1, t2, t3, t4_, t5, t6, t7):
        cid = lax.axis_index("c")
        sid = lax.axis_index("s")
        esets = (e0, e1, e2, e3, e4, e5, e6, e7)
        asets = (a0, a1, a2, a3, a4, a5, a6, a7)
        gbufs = (gb0, gb1)
        sbufs = (sb0, sb1)
        gsems = (g0, g1)
        ssems = (s0, s1)
        stsems = (t0, t1, t2, t3, t4_, t5, t6, t7)

        def fb_pass(j, pcarry):
            fb = cid * (NFB // 2) + j

            def stage_start(q, tc):
                pltpu.async_copy(sd_hbm.at[fb, sid, tc], esets[q], stsems[q])
                pltpu.async_copy(attr_hbm.at[sid, tc], asets[q], stsems[q])

            def stage_wait(q, tc):
                pltpu.make_async_copy(sd_hbm.at[fb, sid, tc], esets[q],
                                      stsems[q]).wait()
                pltpu.make_async_copy(attr_hbm.at[sid, tc], asets[q],
                                      stsems[q]).wait()

            def gather_start(b, q):
                pltpu.async_copy(m_hbm.at[esets[q].at[0]], gbufs[b], gsems[b])

            def gather_wait(b, q):
                pltpu.make_async_copy(m_hbm.at[esets[q].at[0]], gbufs[b],
                                      gsems[b]).wait()

            def scatter_start(b, q):
                pltpu.async_copy(sbufs[b], slab.at[esets[q].at[1]], ssems[b],
                                 add=True)

            def scatter_wait(b, q):
                pltpu.make_async_copy(sbufs[b], slab.at[esets[q].at[1]],
                                      ssems[b]).wait()

            def mul(b, q):
                gbuf, sbuf, attr = gbufs[b], sbufs[b], asets[q]

                def quad(e4, carry):
                    base = e4 * 4
                    gbase = (base // 16) * 16
                    av = attr[pl.ds(gbase, 16)]
                    lb = base - gbase
                    for i in range(4):
                        sp = lax.gather(
                            av, jnp.full((16, 1), lb + i, jnp.int32),
                            lax.GatherDimensionNumbers(
                                offset_dims=(), collapsed_slice_dims=(0,),
                                start_index_map=(0,)),
                            (1,),
                            mode=lax.GatherScatterMode.PROMISE_IN_BOUNDS)
                        e = base + i
                        for c in range(D // 16):
                            sl = pl.ds(c * 16, 16)
                            sbuf[e, sl] = gbuf[e, sl] * sp
                    return carry

                lax.fori_loop(0, ECHUNK // 4, quad, 0)

            pltpu.sync_copy(zeros_hbm, slab.at[pl.ds(sid * STRIPE, STRIPE)])
            plsc.subcore_barrier()

            for q in range(6):
                stage_start(q, q)
            for q in (0, 1):
                stage_wait(q, q)
                gather_start(q, q)

            def body(t8, carry):
                for u in range(NSET):
                    b = u % 2
                    t = NSET * t8 + u
                    gather_wait(b, u)
                    if u < 2:
                        @pl.when(t8 > 0)
                        def _():
                            scatter_wait(b, (u + 6) % NSET)
                    else:
                        scatter_wait(b, (u + 6) % NSET)
                    stage_start((u + 6) % NSET, jnp.minimum(t + 6, NCHUNK - 1))
                    mul(b, u)
                    scatter_start(b, u)
                    stage_wait((u + 2) % NSET, jnp.minimum(t + 2, NCHUNK - 1))
                    gather_start(b, (u + 2) % NSET)
                return carry

            lax.fori_loop(0, NCHUNK // NSET, body, 0)
            scatter_wait(0, 6)
            scatter_wait(1, 7)
            gather_wait(0, 0)
            gather_wait(1, 1)
            for q in (2, 3, 4, 5):
                stage_wait(q, NCHUNK - 1)

            plsc.subcore_barrier()
            pltpu.sync_copy(slab.at[pl.ds(sid * STRIPE, STRIPE)],
                            agg_hbm.at[pl.ds(fb * NP + sid * STRIPE, STRIPE)])
            plsc.subcore_barrier()
            return pcarry

        lax.fori_loop(0, NFB // 2, fb_pass, 0)

    return _sc_agg


def _sc_agg_call(m2, sdp, attrp, zerosb):
    return _sc_agg_fn()(m2, sdp, attrp, zerosb)


def _gate_pack(w):
    wt = w.T.reshape(H, 3, H)
    wt = jnp.pad(wt, ((0, HP - H), (0, 0), (0, HP - H)))
    return wt.reshape(HP, 3 * HP)


def _bias_pack(b):
    bp = jnp.pad(b.reshape(3, H), ((0, 0), (0, HP - H))).reshape(3 * HP)
    return jnp.broadcast_to(bp, (8, 3 * HP))


def kernel(x, edge_index, edge_attr, prob, weight, w_ih, w_hh, b_ih, b_hh,
           lin_w, lin_b):
    f32 = jnp.float32
    xp = jnp.pad(x.astype(f32), ((0, NP - N), (0, 0)))
    h = jnp.pad(xp, ((0, 0), (0, HP - D)))
    w0r = jnp.pad(weight[0][:D, :], ((0, 0), (0, HP - H)))
    wnext = {
        1: jnp.pad(weight[1], ((0, HP - H), (0, HP - H))),
        2: jnp.pad(weight[2], ((0, HP - H), (0, HP - H))),
    }
    wih_t = _gate_pack(w_ih)
    whh_t = _gate_pack(w_hh)
    bi = _bias_pack(b_ih)
    bh = _bias_pack(b_hh)
    srcp = jnp.pad(edge_index[0].astype(jnp.int32), (0, EP - E))
    srcp = (srcp[None, :] + (jnp.arange(NFB, dtype=jnp.int32) * NP)[:, None]
            ).reshape(NFB, NTILES, NCHUNK, 1, ECHUNK)
    dstp = jnp.pad(edge_index[1].astype(jnp.int32),
                   (0, EP - E)).reshape(NTILES, NCHUNK, 1, ECHUNK)
    sdp = jnp.concatenate(
        [srcp, jnp.broadcast_to(dstp[None], srcp.shape)], axis=3)
    attrp = jnp.pad(edge_attr.astype(f32),
                    (0, EP - E)).reshape(NTILES, NCHUNK, ECHUNK)
    zerosb = jnp.zeros((STRIPE, D), f32)
    linp = jnp.zeros((HP, D), f32).at[:H, 0].set(lin_w[0])

    m = _mm0(xp, w0r)
    for layer in (1, 2):
        agg = _sc_agg_call(m.reshape(NFB * NP, D), sdp, attrp, zerosb)
        h, m = _gru_next(agg.reshape(NFB, NP, D), h, wih_t, whh_t, bi, bh,
                         wnext[layer])
    agg = _sc_agg_call(m.reshape(NFB * NP, D), sdp, attrp, zerosb)
    out = _gru_final(agg.reshape(NFB, NP, D), h, wih_t, whh_t, bi, bh, linp)
    return out[:N, :1] + lin_b[0]


# speedup vs baseline: 1.5392x; 1.5392x over previous
import functools

import jax
import jax.numpy as jnp
from jax import lax
from jax.experimental import pallas as pl
from jax.experimental.pallas import tpu as pltpu
from jax.experimental.pallas import tpu_sc as plsc

H = 1000
HP = 1024
D = 128
NFB = HP // D
N = 10000
NP = 10240
E = 320000
ECHUNK = 48
NTILES = 16
NCHUNK = 420
EPT = NCHUNK * ECHUNK
EP = EPT * NTILES
SLAB_N = 10112
STRIPE = SLAB_N // NTILES
BN = 256
BN0 = 1024


def _mm0_body(xb, wb, ob):
    ob[...] = jnp.dot(xb[...], wb[...], preferred_element_type=jnp.float32)[None]


def _mm0(xp, w0r):
    return pl.pallas_call(
        _mm0_body,
        grid=(NP // BN0, NFB),
        in_specs=[
            pl.BlockSpec((BN0, D), lambda i, j: (i, 0)),
            pl.BlockSpec((D, D), lambda i, j: (0, j)),
        ],
        out_specs=pl.BlockSpec((1, BN0, D), lambda i, j: (j, i, 0)),
        out_shape=jax.ShapeDtypeStruct((NFB, NP, D), jnp.float32),
    )(xp, w0r)


def _gru_math(gi, gh, hv):
    r = jax.nn.sigmoid(gi[:, :HP] + gh[:, :HP])
    z = jax.nn.sigmoid(gi[:, HP:2 * HP] + gh[:, HP:2 * HP])
    n = jnp.tanh(gi[:, 2 * HP:] + r * gh[:, 2 * HP:])
    return (1.0 - z) * n + z * hv


def _gates(aggb, hb, wih, whh, bi, bh):
    hv = hb[...]
    bf = jnp.bfloat16
    gh = jnp.dot(hv.astype(bf), whh[...],
                 preferred_element_type=jnp.float32) + bh[...][0:1, :]
    wihv = wih[...]
    gi = bi[...][0:1, :]
    for k in range(NFB):
        gi = gi + jnp.dot(aggb[k].astype(bf), wihv[k * D:(k + 1) * D, :],
                          preferred_element_type=jnp.float32)
    return _gru_math(gi, gh, hv), hv


def _gru_next_body(aggb, hb, wih, whh, bi, bh, wn, hob, mob):
    hn, _ = _gates(aggb, hb, wih, whh, bi, bh)
    hob[...] = hn
    wnv = wn[...]
    hnb = hn.astype(jnp.bfloat16)
    for k in range(NFB):
        mob[k] = jnp.dot(hnb, wnv[:, k * D:(k + 1) * D],
                         preferred_element_type=jnp.float32)


def _gru_next(agg3, h, wih, whh, bi, bh, wn):
    return pl.pallas_call(
        _gru_next_body,
        grid=(NP // BN,),
        in_specs=[
            pl.BlockSpec((NFB, BN, D), lambda i: (0, i, 0)),
            pl.BlockSpec((BN, HP), lambda i: (i, 0)),
            pl.BlockSpec((HP, 3 * HP), lambda i: (0, 0)),
            pl.BlockSpec((HP, 3 * HP), lambda i: (0, 0)),
            pl.BlockSpec((8, 3 * HP), lambda i: (0, 0)),
            pl.BlockSpec((8, 3 * HP), lambda i: (0, 0)),
            pl.BlockSpec((HP, HP), lambda i: (0, 0)),
        ],
        out_specs=[
            pl.BlockSpec((BN, HP), lambda i: (i, 0)),
            pl.BlockSpec((NFB, BN, D), lambda i: (0, i, 0)),
        ],
        out_shape=[
            jax.ShapeDtypeStruct((NP, HP), jnp.float32),
            jax.ShapeDtypeStruct((NFB, NP, D), jnp.float32),
        ],
    )(agg3, h, wih, whh, bi, bh, wn)


def _gru_final_body(aggb, hb, wih, whh, bi, bh, lp, ob):
    hn, _ = _gates(aggb, hb, wih, whh, bi, bh)
    ob[...] = jnp.dot(jnp.maximum(hn, 0.0), lp[...],
                      preferred_element_type=jnp.float32)


def _gru_final(agg3, h, wih, whh, bi, bh, lp):
    return pl.pallas_call(
        _gru_final_body,
        grid=(NP // BN,),
        in_specs=[
            pl.BlockSpec((NFB, BN, D), lambda i: (0, i, 0)),
            pl.BlockSpec((BN, HP), lambda i: (i, 0)),
            pl.BlockSpec((HP, 3 * HP), lambda i: (0, 0)),
            pl.BlockSpec((HP, 3 * HP), lambda i: (0, 0)),
            pl.BlockSpec((8, 3 * HP), lambda i: (0, 0)),
            pl.BlockSpec((8, 3 * HP), lambda i: (0, 0)),
            pl.BlockSpec((HP, D), lambda i: (0, 0)),
        ],
        out_specs=pl.BlockSpec((BN, D), lambda i: (i, 0)),
        out_shape=jax.ShapeDtypeStruct((NP, D), jnp.float32),
    )(agg3, h, wih, whh, bi, bh, lp)


@functools.cache
def _sc_agg_fn():
    mesh = plsc.VectorSubcoreMesh(core_axis_name="c", subcore_axis_name="s")

    NSET = 4

    @functools.partial(
        pl.kernel,
        mesh=mesh,
        out_type=jax.ShapeDtypeStruct((NFB * NP, D), jnp.float32),
        scratch_types=[
            pltpu.VMEM_SHARED((SLAB_N, D), jnp.float32),
        ] + [pltpu.VMEM((ECHUNK,), jnp.int32) for _ in range(NSET)]
          + [pltpu.VMEM((ECHUNK,), jnp.int32) for _ in range(NSET)]
          + [pltpu.VMEM((ECHUNK, 16), jnp.float32) for _ in range(NSET)]
          + [pltpu.VMEM((ECHUNK, D), jnp.float32) for _ in range(2)]
          + [pltpu.VMEM((ECHUNK, D), jnp.float32) for _ in range(2)]
          + [pltpu.SemaphoreType.DMA for _ in range(4 + NSET)],
    )
    def _sc_agg(m_hbm, src_hbm, dst_hbm, attr_hbm, agg_hbm,
                slab,
                e0, e1, e2, e3, d0, d1, d2, d3, a0, a1, a2, a3,
                gb0, gb1, sb0, sb1,
                g0, g1, s0, s1, t0, t1, t2, t3):
        cid = lax.axis_index("c")
        sid = lax.axis_index("s")
        esets = (e0, e1, e2, e3)
        dsets = (d0, d1, d2, d3)
        asets = (a0, a1, a2, a3)
        gbufs = (gb0, gb1)
        sbufs = (sb0, sb1)
        gsems = (g0, g1)
        ssems = (s0, s1)
        stsems = (t0, t1, t2, t3)

        def fb_pass(j, pcarry):
            fb = cid * (NFB // 2) + j

            def stage_start(q, tc):
                pltpu.async_copy(src_hbm.at[fb, sid, tc], esets[q], stsems[q])
                pltpu.async_copy(dst_hbm.at[sid, tc], dsets[q], stsems[q])
                pltpu.async_copy(attr_hbm.at[sid, tc], asets[q], stsems[q])

            def stage_wait(q, tc):
                pltpu.make_async_copy(src_hbm.at[fb, sid, tc], esets[q],
                                      stsems[q]).wait()
                pltpu.make_async_copy(dst_hbm.at[sid, tc], dsets[q],
                                      stsems[q]).wait()
                pltpu.make_async_copy(attr_hbm.at[sid, tc], asets[q],
                                      stsems[q]).wait()

            def gather_start(b, q):
                pltpu.async_copy(m_hbm.at[esets[q]], gbufs[b], gsems[b])

            def gather_wait(b, q):
                pltpu.make_async_copy(m_hbm.at[esets[q]], gbufs[b],
                                      gsems[b]).wait()

            def scatter_start(b, q):
                pltpu.async_copy(sbufs[b], slab.at[dsets[q]], ssems[b],
                                 add=True)

            def scatter_wait(b, q):
                pltpu.make_async_copy(sbufs[b], slab.at[dsets[q]],
                                      ssems[b]).wait()

            def mul(b, q):
                gbuf, sbuf, attr = gbufs[b], sbufs[b], asets[q]

                def edge4(e4i, carry):
                    for u4 in range(4):
                        e = e4i * 4 + u4
                        av = attr[e, :]
                        for c in range(D // 16):
                            sl = pl.ds(c * 16, 16)
                            sbuf[e, sl] = gbuf[e, sl] * av
                    return carry

                lax.fori_loop(0, ECHUNK // 4, edge4, 0)

            def zrow(r, carry):
                for c in range(D // 16):
                    sbufs[0][r, pl.ds(c * 16, 16)] = jnp.zeros((16,),
                                                               jnp.float32)
                return carry

            lax.fori_loop(0, ECHUNK, zrow, 0)
            for r0 in range(0, STRIPE - ECHUNK + 1, ECHUNK):
                pltpu.sync_copy(sbufs[0],
                                slab.at[pl.ds(sid * STRIPE + r0, ECHUNK)])
            _rem = STRIPE % ECHUNK
            if _rem:
                pltpu.sync_copy(
                    sbufs[0].at[pl.ds(0, _rem)],
                    slab.at[pl.ds(sid * STRIPE + STRIPE - _rem, _rem)])
            plsc.subcore_barrier()

            for q in (0, 1):
                stage_start(q, q)
                stage_wait(q, q)
                gather_start(q, q)

            def body(t4, carry):
                for u in range(NSET):
                    b = u % 2
                    qc = (u + 2) % NSET
                    t = NSET * t4 + u
                    gather_wait(b, u)
                    if u < 2:
                        @pl.when(t4 > 0)
                        def _():
                            scatter_wait(b, qc)
                    else:
                        scatter_wait(b, qc)
                    tc = jnp.minimum(t + 2, NCHUNK - 1)
                    stage_start(qc, tc)
                    mul(b, u)
                    scatter_start(b, u)
                    stage_wait(qc, tc)
                    gather_start(b, qc)
                return carry

            lax.fori_loop(0, NCHUNK // NSET, body, 0)
            scatter_wait(0, 2)
            scatter_wait(1, 3)
            gather_wait(0, 0)
            gather_wait(1, 1)

            plsc.subcore_barrier()
            pltpu.sync_copy(slab.at[pl.ds(sid * STRIPE, STRIPE)],
                            agg_hbm.at[pl.ds(fb * NP + sid * STRIPE, STRIPE)])
            plsc.subcore_barrier()
            return pcarry

        lax.fori_loop(0, NFB // 2, fb_pass, 0)

    return _sc_agg


def _sc_agg_call(m2, srcp, dstp, attrp):
    return _sc_agg_fn()(m2, srcp, dstp, attrp)


def _gate_pack(w):
    wt = w.T.reshape(H, 3, H)
    wt = jnp.pad(wt, ((0, HP - H), (0, 0), (0, HP - H)))
    return wt.reshape(HP, 3 * HP)


def _bias_pack(b):
    bp = jnp.pad(b.reshape(3, H), ((0, 0), (0, HP - H))).reshape(3 * HP)
    return jnp.broadcast_to(bp, (8, 3 * HP))


def kernel(x, edge_index, edge_attr, prob, weight, w_ih, w_hh, b_ih, b_hh,
           lin_w, lin_b):
    f32 = jnp.float32
    xp = jnp.pad(x.astype(f32), ((0, NP - N), (0, 0)))
    h = jnp.pad(xp, ((0, 0), (0, HP - D)))
    w0r = jnp.pad(weight[0][:D, :], ((0, 0), (0, HP - H)))
    wnext = {
        1: jnp.pad(weight[1], ((0, HP - H), (0, HP - H))).astype(jnp.bfloat16),
        2: jnp.pad(weight[2], ((0, HP - H), (0, HP - H))).astype(jnp.bfloat16),
    }
    wih_t = _gate_pack(w_ih).astype(jnp.bfloat16)
    whh_t = _gate_pack(w_hh).astype(jnp.bfloat16)
    bi = _bias_pack(b_ih)
    bh = _bias_pack(b_hh)
    srcp = jnp.pad(edge_index[0].astype(jnp.int32), (0, EP - E))
    srcp = (srcp[None, :] + (jnp.arange(NFB, dtype=jnp.int32) * NP)[:, None]
            ).reshape(NFB, NTILES, NCHUNK, ECHUNK)
    dstp = jnp.pad(edge_index[1].astype(jnp.int32),
                   (0, EP - E)).reshape(NTILES, NCHUNK, ECHUNK)
    attrp = jnp.pad(edge_attr.astype(f32), (0, EP - E))
    attrp = jnp.broadcast_to(attrp[:, None],
                             (EP, 16)).reshape(NTILES, NCHUNK, ECHUNK, 16)
    linp = jnp.zeros((HP, D), f32).at[:H, 0].set(lin_w[0])

    m = _mm0(xp, w0r)
    for layer in (1, 2):
        agg = _sc_agg_call(m.reshape(NFB * NP, D), srcp, dstp, attrp)
        h, m = _gru_next(agg.reshape(NFB, NP, D), h, wih_t, whh_t, bi, bh,
                         wnext[layer])
    agg = _sc_agg_call(m.reshape(NFB * NP, D), srcp, dstp, attrp)
    out = _gru_final(agg.reshape(NFB, NP, D), h, wih_t, whh_t, bi, bh, linp)
    return out[:N, :1] + lin_b[0]


# speedup vs baseline: 1.5568x; 1.0114x over previous
import functools

import jax
import jax.numpy as jnp
from jax import lax
from jax.experimental import pallas as pl
from jax.experimental.pallas import tpu as pltpu
from jax.experimental.pallas import tpu_sc as plsc

H = 1000
HP = 1024
D = 128
NFB = HP // D
N = 10000
NP = 10240
E = 320000
ECHUNK = 48
NTILES = 16
NCHUNK = 420
EPT = NCHUNK * ECHUNK
EP = EPT * NTILES
SLAB_N = 10112
STRIPE = SLAB_N // NTILES
BN = 256
BN0 = 1024


def _mm0_body(xb, wb, ob):
    ob[...] = jnp.dot(xb[...], wb[...], preferred_element_type=jnp.float32)[None]


def _mm0(xp, w0r):
    return pl.pallas_call(
        _mm0_body,
        grid=(NP // BN0, NFB),
        in_specs=[
            pl.BlockSpec((BN0, D), lambda i, j: (i, 0)),
            pl.BlockSpec((D, D), lambda i, j: (0, j)),
        ],
        out_specs=pl.BlockSpec((1, BN0, D), lambda i, j: (j, i, 0)),
        out_shape=jax.ShapeDtypeStruct((NFB, NP, D), jnp.float32),
    )(xp, w0r)


def _gru_math(gi, gh, hv):
    r = jax.nn.sigmoid(gi[:, :HP] + gh[:, :HP])
    z = jax.nn.sigmoid(gi[:, HP:2 * HP] + gh[:, HP:2 * HP])
    n = jnp.tanh(gi[:, 2 * HP:] + r * gh[:, 2 * HP:])
    return (1.0 - z) * n + z * hv


def _gates(aggb, hb, wih, whh, bi, bh):
    hv = hb[...]
    bf = jnp.bfloat16
    gh = jnp.dot(hv.astype(bf), whh[...],
                 preferred_element_type=jnp.float32) + bh[...][0:1, :]
    wihv = wih[...]
    gi = bi[...][0:1, :]
    for k in range(NFB):
        gi = gi + jnp.dot(aggb[k].astype(bf), wihv[k * D:(k + 1) * D, :],
                          preferred_element_type=jnp.float32)
    return _gru_math(gi, gh, hv), hv


def _gru_next_body(aggb, hb, wih, whh, bi, bh, wn, hob, mob):
    hn, _ = _gates(aggb, hb, wih, whh, bi, bh)
    hob[...] = hn
    wnv = wn[...]
    hnb = hn.astype(jnp.bfloat16)
    for k in range(NFB):
        mob[k] = jnp.dot(hnb, wnv[:, k * D:(k + 1) * D],
                         preferred_element_type=jnp.float32)


def _gru_next(agg3, h, wih, whh, bi, bh, wn):
    return pl.pallas_call(
        _gru_next_body,
        grid=(NP // BN,),
        in_specs=[
            pl.BlockSpec((NFB, BN, D), lambda i: (0, i, 0)),
            pl.BlockSpec((BN, HP), lambda i: (i, 0)),
            pl.BlockSpec((HP, 3 * HP), lambda i: (0, 0)),
            pl.BlockSpec((HP, 3 * HP), lambda i: (0, 0)),
            pl.BlockSpec((8, 3 * HP), lambda i: (0, 0)),
            pl.BlockSpec((8, 3 * HP), lambda i: (0, 0)),
            pl.BlockSpec((HP, HP), lambda i: (0, 0)),
        ],
        out_specs=[
            pl.BlockSpec((BN, HP), lambda i: (i, 0)),
            pl.BlockSpec((NFB, BN, D), lambda i: (0, i, 0)),
        ],
        out_shape=[
            jax.ShapeDtypeStruct((NP, HP), jnp.float32),
            jax.ShapeDtypeStruct((NFB, NP, D), jnp.float32),
        ],
    )(agg3, h, wih, whh, bi, bh, wn)


def _gru_final_body(aggb, hb, wih, whh, bi, bh, lp, ob):
    hn, _ = _gates(aggb, hb, wih, whh, bi, bh)
    ob[...] = jnp.dot(jnp.maximum(hn, 0.0), lp[...],
                      preferred_element_type=jnp.float32)


def _gru_final(agg3, h, wih, whh, bi, bh, lp):
    return pl.pallas_call(
        _gru_final_body,
        grid=(NP // BN,),
        in_specs=[
            pl.BlockSpec((NFB, BN, D), lambda i: (0, i, 0)),
            pl.BlockSpec((BN, HP), lambda i: (i, 0)),
            pl.BlockSpec((HP, 3 * HP), lambda i: (0, 0)),
            pl.BlockSpec((HP, 3 * HP), lambda i: (0, 0)),
            pl.BlockSpec((8, 3 * HP), lambda i: (0, 0)),
            pl.BlockSpec((8, 3 * HP), lambda i: (0, 0)),
            pl.BlockSpec((HP, D), lambda i: (0, 0)),
        ],
        out_specs=pl.BlockSpec((BN, D), lambda i: (i, 0)),
        out_shape=jax.ShapeDtypeStruct((NP, D), jnp.float32),
    )(agg3, h, wih, whh, bi, bh, lp)


@functools.cache
def _sc_agg_fn():
    mesh = plsc.VectorSubcoreMesh(core_axis_name="c", subcore_axis_name="s")

    NSET = 4

    @functools.partial(
        pl.kernel,
        mesh=mesh,
        out_type=jax.ShapeDtypeStruct((NFB * NP, D), jnp.float32),
        scratch_types=[
            pltpu.VMEM_SHARED((SLAB_N, D), jnp.float32),
        ] + [pltpu.VMEM((ECHUNK,), jnp.int32) for _ in range(NSET)]
          + [pltpu.VMEM((ECHUNK,), jnp.int32) for _ in range(NSET)]
          + [pltpu.VMEM((ECHUNK, 16), jnp.float32) for _ in range(NSET)]
          + [pltpu.VMEM((ECHUNK, D), jnp.float32) for _ in range(2)]
          + [pltpu.VMEM((ECHUNK, D), jnp.float32) for _ in range(2)]
          + [pltpu.SemaphoreType.DMA for _ in range(4 + NSET)],
    )
    def _sc_agg(m_hbm, src_hbm, dst_hbm, attr_hbm, agg_hbm,
                slab,
                e0, e1, e2, e3, d0, d1, d2, d3, a0, a1, a2, a3,
                gb0, gb1, sb0, sb1,
                g0, g1, s0, s1, t0, t1, t2, t3):
        cid = lax.axis_index("c")
        sid = lax.axis_index("s")
        esets = (e0, e1, e2, e3)
        dsets = (d0, d1, d2, d3)
        asets = (a0, a1, a2, a3)
        gbufs = (gb0, gb1)
        sbufs = (sb0, sb1)
        gsems = (g0, g1)
        ssems = (s0, s1)
        stsems = (t0, t1, t2, t3)

        def fb_pass(j, pcarry):
            fb = cid * (NFB // 2) + j

            def stage_start(q, tc):
                pltpu.async_copy(src_hbm.at[fb, sid, tc], esets[q], stsems[q])
                pltpu.async_copy(dst_hbm.at[sid, tc], dsets[q], stsems[q])
                pltpu.async_copy(attr_hbm.at[sid, tc], asets[q], stsems[q])

            def stage_wait(q, tc):
                pltpu.make_async_copy(src_hbm.at[fb, sid, tc], esets[q],
                                      stsems[q]).wait()
                pltpu.make_async_copy(dst_hbm.at[sid, tc], dsets[q],
                                      stsems[q]).wait()
                pltpu.make_async_copy(attr_hbm.at[sid, tc], asets[q],
                                      stsems[q]).wait()

            def gather_start(b, q):
                pltpu.async_copy(m_hbm.at[esets[q]], gbufs[b], gsems[b])

            def gather_wait(b, q):
                pltpu.make_async_copy(m_hbm.at[esets[q]], gbufs[b],
                                      gsems[b]).wait()

            def scatter_start(b, q):
                pltpu.async_copy(sbufs[b], slab.at[dsets[q]], ssems[b],
                                 add=True)

            def scatter_wait(b, q):
                pltpu.make_async_copy(sbufs[b], slab.at[dsets[q]],
                                      ssems[b]).wait()

            def mul(b, q):
                gbuf, sbuf, attr = gbufs[b], sbufs[b], asets[q]

                def edge4(e4i, carry):
                    for u4 in range(4):
                        e = e4i * 4 + u4
                        av = attr[e, :]
                        for c in range(D // 16):
                            sl = pl.ds(c * 16, 16)
                            sbuf[e, sl] = gbuf[e, sl] * av
                    return carry

                lax.fori_loop(0, ECHUNK // 4, edge4, 0)

            def zrow(r, carry):
                for c in range(D // 16):
                    sbufs[0][r, pl.ds(c * 16, 16)] = jnp.zeros((16,),
                                                               jnp.float32)
                return carry

            lax.fori_loop(0, ECHUNK, zrow, 0)
            for r0 in range(0, STRIPE - ECHUNK + 1, ECHUNK):
                pltpu.sync_copy(sbufs[0],
                                slab.at[pl.ds(sid * STRIPE + r0, ECHUNK)])
            _rem = STRIPE % ECHUNK
            if _rem:
                pltpu.sync_copy(
                    sbufs[0].at[pl.ds(0, _rem)],
                    slab.at[pl.ds(sid * STRIPE + STRIPE - _rem, _rem)])
            plsc.subcore_barrier()

            for q in (0, 1):
                stage_start(q, q)
                stage_wait(q, q)
                gather_start(q, q)

            def body(t4, carry):
                for u in range(NSET):
                    b = u % 2
                    qc = (u + 2) % NSET
                    t = NSET * t4 + u
                    if u < 2:
                        @pl.when(t4 > 0)
                        def _():
                            scatter_wait(b, qc)
                    else:
                        scatter_wait(b, qc)
                    tc = jnp.minimum(t + 2, NCHUNK - 1)
                    stage_start(qc, tc)
                    gather_wait(b, u)
                    mul(b, u)
                    scatter_start(b, u)
                    stage_wait(qc, tc)
                    gather_start(b, qc)
                return carry

            lax.fori_loop(0, NCHUNK // NSET, body, 0)
            scatter_wait(0, 2)
            scatter_wait(1, 3)
            gather_wait(0, 0)
            gather_wait(1, 1)

            plsc.subcore_barrier()
            pltpu.sync_copy(slab.at[pl.ds(sid * STRIPE, STRIPE)],
                            agg_hbm.at[pl.ds(fb * NP + sid * STRIPE, STRIPE)])
            plsc.subcore_barrier()
            return pcarry

        lax.fori_loop(0, NFB // 2, fb_pass, 0)

    return _sc_agg


def _sc_agg_call(m2, srcp, dstp, attrp):
    return _sc_agg_fn()(m2, srcp, dstp, attrp)


def _gate_pack(w):
    wt = w.T.reshape(H, 3, H)
    wt = jnp.pad(wt, ((0, HP - H), (0, 0), (0, HP - H)))
    return wt.reshape(HP, 3 * HP)


def _bias_pack(b):
    bp = jnp.pad(b.reshape(3, H), ((0, 0), (0, HP - H))).reshape(3 * HP)
    return jnp.broadcast_to(bp, (8, 3 * HP))


def kernel(x, edge_index, edge_attr, prob, weight, w_ih, w_hh, b_ih, b_hh,
           lin_w, lin_b):
    f32 = jnp.float32
    xp = jnp.pad(x.astype(f32), ((0, NP - N), (0, 0)))
    h = jnp.pad(xp, ((0, 0), (0, HP - D)))
    w0r = jnp.pad(weight[0][:D, :], ((0, 0), (0, HP - H)))
    wnext = {
        1: jnp.pad(weight[1], ((0, HP - H), (0, HP - H))).astype(jnp.bfloat16),
        2: jnp.pad(weight[2], ((0, HP - H), (0, HP - H))).astype(jnp.bfloat16),
    }
    wih_t = _gate_pack(w_ih).astype(jnp.bfloat16)
    whh_t = _gate_pack(w_hh).astype(jnp.bfloat16)
    bi = _bias_pack(b_ih)
    bh = _bias_pack(b_hh)
    srcp = jnp.pad(edge_index[0].astype(jnp.int32), (0, EP - E))
    srcp = (srcp[None, :] + (jnp.arange(NFB, dtype=jnp.int32) * NP)[:, None]
            ).reshape(NFB, NTILES, NCHUNK, ECHUNK)
    dstp = jnp.pad(edge_index[1].astype(jnp.int32),
                   (0, EP - E)).reshape(NTILES, NCHUNK, ECHUNK)
    attrp = jnp.pad(edge_attr.astype(f32), (0, EP - E))
    attrp = jnp.broadcast_to(attrp[:, None],
                             (EP, 16)).reshape(NTILES, NCHUNK, ECHUNK, 16)
    linp = jnp.zeros((HP, D), f32).at[:H, 0].set(lin_w[0])

    m = _mm0(xp, w0r)
    for layer in (1, 2):
        agg = _sc_agg_call(m.reshape(NFB * NP, D), srcp, dstp, attrp)
        h, m = _gru_next(agg.reshape(NFB, NP, D), h, wih_t, whh_t, bi, bh,
                         wnext[layer])
    agg = _sc_agg_call(m.reshape(NFB * NP, D), srcp, dstp, attrp)
    out = _gru_final(agg.reshape(NFB, NP, D), h, wih_t, whh_t, bi, bh, linp)
    return out[:N, :1] + lin_b[0]


# speedup vs baseline: 1.5653x; 1.0055x over previous
import functools

import jax
import jax.numpy as jnp
from jax import lax
from jax.experimental import pallas as pl
from jax.experimental.pallas import tpu as pltpu
from jax.experimental.pallas import tpu_sc as plsc

H = 1000
HP = 1024
D = 128
NFB = HP // D
N = 10000
NP = 10240
E = 320000
ECHUNK = 48
NTILES = 16
NCHUNK = 420
EPT = NCHUNK * ECHUNK
EP = EPT * NTILES
SLAB_N = 10112
STRIPE = SLAB_N // NTILES
BN = 512
BN0 = 1024


def _mm0_body(xb, wb, ob):
    ob[...] = jnp.dot(xb[...], wb[...], preferred_element_type=jnp.float32)[None]


def _mm0(xp, w0r):
    return pl.pallas_call(
        _mm0_body,
        grid=(NP // BN0, NFB),
        in_specs=[
            pl.BlockSpec((BN0, D), lambda i, j: (i, 0)),
            pl.BlockSpec((D, D), lambda i, j: (0, j)),
        ],
        out_specs=pl.BlockSpec((1, BN0, D), lambda i, j: (j, i, 0)),
        out_shape=jax.ShapeDtypeStruct((NFB, NP, D), jnp.float32),
    )(xp, w0r)


def _gru_math(gi, gh, hv):
    r = jax.nn.sigmoid(gi[:, :HP] + gh[:, :HP])
    z = jax.nn.sigmoid(gi[:, HP:2 * HP] + gh[:, HP:2 * HP])
    n = jnp.tanh(gi[:, 2 * HP:] + r * gh[:, 2 * HP:])
    return (1.0 - z) * n + z * hv


def _gates(aggb, hb, wih, whh, bi, bh):
    hv = hb[...]
    bf = jnp.bfloat16
    gh = jnp.dot(hv.astype(bf), whh[...],
                 preferred_element_type=jnp.float32) + bh[...][0:1, :]
    wihv = wih[...]
    gi = bi[...][0:1, :]
    for k in range(NFB):
        gi = gi + jnp.dot(aggb[k].astype(bf), wihv[k * D:(k + 1) * D, :],
                          preferred_element_type=jnp.float32)
    return _gru_math(gi, gh, hv), hv


def _gru_next_body(aggb, hb, wih, whh, bi, bh, wn, hob, mob):
    hn, _ = _gates(aggb, hb, wih, whh, bi, bh)
    hob[...] = hn
    wnv = wn[...]
    hnb = hn.astype(jnp.bfloat16)
    for k in range(NFB):
        mob[k] = jnp.dot(hnb, wnv[:, k * D:(k + 1) * D],
                         preferred_element_type=jnp.float32)


def _gru_next(agg3, h, wih, whh, bi, bh, wn):
    return pl.pallas_call(
        _gru_next_body,
        grid=(NP // BN,),
        in_specs=[
            pl.BlockSpec((NFB, BN, D), lambda i: (0, i, 0)),
            pl.BlockSpec((BN, HP), lambda i: (i, 0)),
            pl.BlockSpec((HP, 3 * HP), lambda i: (0, 0)),
            pl.BlockSpec((HP, 3 * HP), lambda i: (0, 0)),
            pl.BlockSpec((8, 3 * HP), lambda i: (0, 0)),
            pl.BlockSpec((8, 3 * HP), lambda i: (0, 0)),
            pl.BlockSpec((HP, HP), lambda i: (0, 0)),
        ],
        out_specs=[
            pl.BlockSpec((BN, HP), lambda i: (i, 0)),
            pl.BlockSpec((NFB, BN, D), lambda i: (0, i, 0)),
        ],
        out_shape=[
            jax.ShapeDtypeStruct((NP, HP), jnp.float32),
            jax.ShapeDtypeStruct((NFB, NP, D), jnp.float32),
        ],
    )(agg3, h, wih, whh, bi, bh, wn)


def _gru_final_body(aggb, hb, wih, whh, bi, bh, lp, ob):
    hn, _ = _gates(aggb, hb, wih, whh, bi, bh)
    ob[...] = jnp.dot(jnp.maximum(hn, 0.0), lp[...],
                      preferred_element_type=jnp.float32)


def _gru_final(agg3, h, wih, whh, bi, bh, lp):
    return pl.pallas_call(
        _gru_final_body,
        grid=(NP // BN,),
        in_specs=[
            pl.BlockSpec((NFB, BN, D), lambda i: (0, i, 0)),
            pl.BlockSpec((BN, HP), lambda i: (i, 0)),
            pl.BlockSpec((HP, 3 * HP), lambda i: (0, 0)),
            pl.BlockSpec((HP, 3 * HP), lambda i: (0, 0)),
            pl.BlockSpec((8, 3 * HP), lambda i: (0, 0)),
            pl.BlockSpec((8, 3 * HP), lambda i: (0, 0)),
            pl.BlockSpec((HP, D), lambda i: (0, 0)),
        ],
        out_specs=pl.BlockSpec((BN, D), lambda i: (i, 0)),
        out_shape=jax.ShapeDtypeStruct((NP, D), jnp.float32),
    )(agg3, h, wih, whh, bi, bh, lp)


@functools.cache
def _sc_agg_fn():
    mesh = plsc.VectorSubcoreMesh(core_axis_name="c", subcore_axis_name="s")

    NSET = 4

    @functools.partial(
        pl.kernel,
        mesh=mesh,
        out_type=jax.ShapeDtypeStruct((NFB * NP, D), jnp.float32),
        scratch_types=[
            pltpu.VMEM_SHARED((SLAB_N, D), jnp.float32),
        ] + [pltpu.VMEM((ECHUNK,), jnp.int32) for _ in range(NSET)]
          + [pltpu.VMEM((ECHUNK,), jnp.int32) for _ in range(NSET)]
          + [pltpu.VMEM((ECHUNK, 16), jnp.float32) for _ in range(NSET)]
          + [pltpu.VMEM((ECHUNK, D), jnp.float32) for _ in range(2)]
          + [pltpu.VMEM((ECHUNK, D), jnp.float32) for _ in range(2)]
          + [pltpu.SemaphoreType.DMA for _ in range(4 + NSET)],
    )
    def _sc_agg(m_hbm, src_hbm, dst_hbm, attr_hbm, agg_hbm,
                slab,
                e0, e1, e2, e3, d0, d1, d2, d3, a0, a1, a2, a3,
                gb0, gb1, sb0, sb1,
                g0, g1, s0, s1, t0, t1, t2, t3):
        cid = lax.axis_index("c")
        sid = lax.axis_index("s")
        esets = (e0, e1, e2, e3)
        dsets = (d0, d1, d2, d3)
        asets = (a0, a1, a2, a3)
        gbufs = (gb0, gb1)
        sbufs = (sb0, sb1)
        gsems = (g0, g1)
        ssems = (s0, s1)
        stsems = (t0, t1, t2, t3)

        def fb_pass(j, pcarry):
            fb = cid * (NFB // 2) + j

            def stage_start(q, tc):
                pltpu.async_copy(src_hbm.at[fb, sid, tc], esets[q], stsems[q])
                pltpu.async_copy(dst_hbm.at[sid, tc], dsets[q], stsems[q])
                pltpu.async_copy(attr_hbm.at[sid, tc], asets[q], stsems[q])

            def stage_wait(q, tc):
                pltpu.make_async_copy(src_hbm.at[fb, sid, tc], esets[q],
                                      stsems[q]).wait()
                pltpu.make_async_copy(dst_hbm.at[sid, tc], dsets[q],
                                      stsems[q]).wait()
                pltpu.make_async_copy(attr_hbm.at[sid, tc], asets[q],
                                      stsems[q]).wait()

            def gather_start(b, q):
                pltpu.async_copy(m_hbm.at[esets[q]], gbufs[b], gsems[b])

            def gather_wait(b, q):
                pltpu.make_async_copy(m_hbm.at[esets[q]], gbufs[b],
                                      gsems[b]).wait()

            def scatter_start(b, q):
                pltpu.async_copy(sbufs[b], slab.at[dsets[q]], ssems[b],
                                 add=True)

            def scatter_wait(b, q):
                pltpu.make_async_copy(sbufs[b], slab.at[dsets[q]],
                                      ssems[b]).wait()

            def mul(b, q):
                gbuf, sbuf, attr = gbufs[b], sbufs[b], asets[q]

                def edge4(e4i, carry):
                    for u4 in range(4):
                        e = e4i * 4 + u4
                        av = attr[e, :]
                        for c in range(D // 16):
                            sl = pl.ds(c * 16, 16)
                            sbuf[e, sl] = gbuf[e, sl] * av
                    return carry

                lax.fori_loop(0, ECHUNK // 4, edge4, 0)

            def zrow(r, carry):
                for c in range(D // 16):
                    sbufs[0][r, pl.ds(c * 16, 16)] = jnp.zeros((16,),
                                                               jnp.float32)
                return carry

            lax.fori_loop(0, ECHUNK, zrow, 0)
            for r0 in range(0, STRIPE - ECHUNK + 1, ECHUNK):
                pltpu.sync_copy(sbufs[0],
                                slab.at[pl.ds(sid * STRIPE + r0, ECHUNK)])
            _rem = STRIPE % ECHUNK
            if _rem:
                pltpu.sync_copy(
                    sbufs[0].at[pl.ds(0, _rem)],
                    slab.at[pl.ds(sid * STRIPE + STRIPE - _rem, _rem)])
            plsc.subcore_barrier()

            for q in (0, 1):
                stage_start(q, q)
                stage_wait(q, q)
                gather_start(q, q)

            def body(t4, carry):
                for u in range(NSET):
                    b = u % 2
                    qc = (u + 2) % NSET
                    t = NSET * t4 + u
                    if u < 2:
                        @pl.when(t4 > 0)
                        def _():
                            scatter_wait(b, qc)
                    else:
                        scatter_wait(b, qc)
                    tc = jnp.minimum(t + 2, NCHUNK - 1)
                    stage_start(qc, tc)
                    gather_wait(b, u)
                    mul(b, u)
                    scatter_start(b, u)
                    stage_wait(qc, tc)
                    gather_start(b, qc)
                return carry

            lax.fori_loop(0, NCHUNK // NSET, body, 0)
            scatter_wait(0, 2)
            scatter_wait(1, 3)
            gather_wait(0, 0)
            gather_wait(1, 1)

            plsc.subcore_barrier()
            pltpu.sync_copy(slab.at[pl.ds(sid * STRIPE, STRIPE)],
                            agg_hbm.at[pl.ds(fb * NP + sid * STRIPE, STRIPE)])
            plsc.subcore_barrier()
            return pcarry

        lax.fori_loop(0, NFB // 2, fb_pass, 0)

    return _sc_agg


def _sc_agg_call(m2, srcp, dstp, attrp):
    return _sc_agg_fn()(m2, srcp, dstp, attrp)


def _gate_pack(w):
    wt = w.T.reshape(H, 3, H)
    wt = jnp.pad(wt, ((0, HP - H), (0, 0), (0, HP - H)))
    return wt.reshape(HP, 3 * HP)


def _bias_pack(b):
    bp = jnp.pad(b.reshape(3, H), ((0, 0), (0, HP - H))).reshape(3 * HP)
    return jnp.broadcast_to(bp, (8, 3 * HP))


def kernel(x, edge_index, edge_attr, prob, weight, w_ih, w_hh, b_ih, b_hh,
           lin_w, lin_b):
    f32 = jnp.float32
    xp = jnp.pad(x.astype(f32), ((0, NP - N), (0, 0)))
    h = jnp.pad(xp, ((0, 0), (0, HP - D)))
    w0r = jnp.pad(weight[0][:D, :], ((0, 0), (0, HP - H)))
    wnext = {
        1: jnp.pad(weight[1], ((0, HP - H), (0, HP - H))).astype(jnp.bfloat16),
        2: jnp.pad(weight[2], ((0, HP - H), (0, HP - H))).astype(jnp.bfloat16),
    }
    wih_t = _gate_pack(w_ih).astype(jnp.bfloat16)
    whh_t = _gate_pack(w_hh).astype(jnp.bfloat16)
    bi = _bias_pack(b_ih)
    bh = _bias_pack(b_hh)
    srcp = jnp.pad(edge_index[0].astype(jnp.int32), (0, EP - E))
    srcp = (srcp[None, :] + (jnp.arange(NFB, dtype=jnp.int32) * NP)[:, None]
            ).reshape(NFB, NTILES, NCHUNK, ECHUNK)
    dstp = jnp.pad(edge_index[1].astype(jnp.int32),
                   (0, EP - E)).reshape(NTILES, NCHUNK, ECHUNK)
    attrp = jnp.pad(edge_attr.astype(f32), (0, EP - E))
    attrp = jnp.broadcast_to(attrp[:, None],
                             (EP, 16)).reshape(NTILES, NCHUNK, ECHUNK, 16)
    linp = jnp.zeros((HP, D), f32).at[:H, 0].set(lin_w[0])

    m = _mm0(xp, w0r)
    for layer in (1, 2):
        agg = _sc_agg_call(m.reshape(NFB * NP, D), srcp, dstp, attrp)
        h, m = _gru_next(agg.reshape(NFB, NP, D), h, wih_t, whh_t, bi, bh,
                         wnext[layer])
    agg = _sc_agg_call(m.reshape(NFB * NP, D), srcp, dstp, attrp)
    out = _gru_final(agg.reshape(NFB, NP, D), h, wih_t, whh_t, bi, bh, linp)
    return out[:N, :1] + lin_b[0]


# speedup vs baseline: 1.6025x; 1.0238x over previous
import functools

import jax
import jax.numpy as jnp
from jax import lax
from jax.experimental import pallas as pl
from jax.experimental.pallas import tpu as pltpu
from jax.experimental.pallas import tpu_sc as plsc

H = 1000
HP = 1024
D = 128
NFB = HP // D
N = 10000
NP = 10240
E = 320000
ECHUNK = 48
NTILES = 16
NCHUNK = 420
EPT = NCHUNK * ECHUNK
EP = EPT * NTILES
SLAB_N = 10112
STRIPE = SLAB_N // NTILES
BN = 512
BN0 = 1024


def _mm0_body(xb, wb, ob):
    ob[...] = jnp.dot(xb[...], wb[...], preferred_element_type=jnp.float32)[None]


def _mm0(xp, w0r):
    return pl.pallas_call(
        _mm0_body,
        grid=(NP // BN0, NFB),
        in_specs=[
            pl.BlockSpec((BN0, D), lambda i, j: (i, 0)),
            pl.BlockSpec((D, D), lambda i, j: (0, j)),
        ],
        out_specs=pl.BlockSpec((1, BN0, D), lambda i, j: (j, i, 0)),
        out_shape=jax.ShapeDtypeStruct((NFB, NP, D), jnp.float32),
    )(xp, w0r)


def _gh_body(hb, whh, bh, ob):
    ob[...] = (jnp.dot(hb[...].astype(jnp.bfloat16), whh[...],
                       preferred_element_type=jnp.float32)
               + bh[...][0:1, :]).astype(jnp.bfloat16)


def _gh(h, whh, bh):
    return pl.pallas_call(
        _gh_body,
        grid=(NP // BN,),
        in_specs=[
            pl.BlockSpec((BN, HP), lambda i: (i, 0)),
            pl.BlockSpec((HP, 3 * HP), lambda i: (0, 0)),
            pl.BlockSpec((8, 3 * HP), lambda i: (0, 0)),
        ],
        out_specs=pl.BlockSpec((BN, 3 * HP), lambda i: (i, 0)),
        out_shape=jax.ShapeDtypeStruct((NP, 3 * HP), jnp.bfloat16),
    )(h, whh, bh)


def _gru_math(gi, gh, hv):
    r = jax.nn.sigmoid(gi[:, :HP] + gh[:, :HP])
    z = jax.nn.sigmoid(gi[:, HP:2 * HP] + gh[:, HP:2 * HP])
    n = jnp.tanh(gi[:, 2 * HP:] + r * gh[:, 2 * HP:])
    return (1.0 - z) * n + z * hv


def _gates(aggb, hb, ghb, wih, bi):
    hv = hb[...]
    bf = jnp.bfloat16
    gh = ghb[...].astype(jnp.float32)
    wihv = wih[...]
    gi = bi[...][0:1, :]
    for k in range(NFB):
        gi = gi + jnp.dot(aggb[k].astype(bf), wihv[k * D:(k + 1) * D, :],
                          preferred_element_type=jnp.float32)
    return _gru_math(gi, gh, hv), hv


def _gru_next_body(aggb, hb, ghb, wih, bi, wn, hob, mob):
    hn, _ = _gates(aggb, hb, ghb, wih, bi)
    hob[...] = hn
    wnv = wn[...]
    hnb = hn.astype(jnp.bfloat16)
    for k in range(NFB):
        mob[k] = jnp.dot(hnb, wnv[:, k * D:(k + 1) * D],
                         preferred_element_type=jnp.float32)


def _gru_next(agg3, h, gh, wih, bi, wn):
    return pl.pallas_call(
        _gru_next_body,
        grid=(NP // BN,),
        in_specs=[
            pl.BlockSpec((NFB, BN, D), lambda i: (0, i, 0)),
            pl.BlockSpec((BN, HP), lambda i: (i, 0)),
            pl.BlockSpec((BN, 3 * HP), lambda i: (i, 0)),
            pl.BlockSpec((HP, 3 * HP), lambda i: (0, 0)),
            pl.BlockSpec((8, 3 * HP), lambda i: (0, 0)),
            pl.BlockSpec((HP, HP), lambda i: (0, 0)),
        ],
        out_specs=[
            pl.BlockSpec((BN, HP), lambda i: (i, 0)),
            pl.BlockSpec((NFB, BN, D), lambda i: (0, i, 0)),
        ],
        out_shape=[
            jax.ShapeDtypeStruct((NP, HP), jnp.float32),
            jax.ShapeDtypeStruct((NFB, NP, D), jnp.float32),
        ],
    )(agg3, h, gh, wih, bi, wn)


def _gru_final_body(aggb, hb, ghb, wih, bi, lp, ob):
    hn, _ = _gates(aggb, hb, ghb, wih, bi)
    ob[...] = jnp.dot(jnp.maximum(hn, 0.0), lp[...],
                      preferred_element_type=jnp.float32)


def _gru_final(agg3, h, gh, wih, bi, lp):
    return pl.pallas_call(
        _gru_final_body,
        grid=(NP // BN,),
        in_specs=[
            pl.BlockSpec((NFB, BN, D), lambda i: (0, i, 0)),
            pl.BlockSpec((BN, HP), lambda i: (i, 0)),
            pl.BlockSpec((BN, 3 * HP), lambda i: (i, 0)),
            pl.BlockSpec((HP, 3 * HP), lambda i: (0, 0)),
            pl.BlockSpec((8, 3 * HP), lambda i: (0, 0)),
            pl.BlockSpec((HP, D), lambda i: (0, 0)),
        ],
        out_specs=pl.BlockSpec((BN, D), lambda i: (i, 0)),
        out_shape=jax.ShapeDtypeStruct((NP, D), jnp.float32),
    )(agg3, h, gh, wih, bi, lp)


@functools.cache
def _sc_agg_fn():
    mesh = plsc.VectorSubcoreMesh(core_axis_name="c", subcore_axis_name="s")

    NSET = 4

    @functools.partial(
        pl.kernel,
        mesh=mesh,
        out_type=jax.ShapeDtypeStruct((NFB * NP, D), jnp.float32),
        scratch_types=[
            pltpu.VMEM_SHARED((SLAB_N, D), jnp.float32),
        ] + [pltpu.VMEM((ECHUNK,), jnp.int32) for _ in range(NSET)]
          + [pltpu.VMEM((ECHUNK,), jnp.int32) for _ in range(NSET)]
          + [pltpu.VMEM((ECHUNK, 16), jnp.float32) for _ in range(NSET)]
          + [pltpu.VMEM((ECHUNK, D), jnp.float32) for _ in range(2)]
          + [pltpu.VMEM((ECHUNK, D), jnp.float32) for _ in range(2)]
          + [pltpu.SemaphoreType.DMA for _ in range(4 + NSET)],
    )
    def _sc_agg(m_hbm, src_hbm, dst_hbm, attr_hbm, agg_hbm,
                slab,
                e0, e1, e2, e3, d0, d1, d2, d3, a0, a1, a2, a3,
                gb0, gb1, sb0, sb1,
                g0, g1, s0, s1, t0, t1, t2, t3):
        cid = lax.axis_index("c")
        sid = lax.axis_index("s")
        esets = (e0, e1, e2, e3)
        dsets = (d0, d1, d2, d3)
        asets = (a0, a1, a2, a3)
        gbufs = (gb0, gb1)
        sbufs = (sb0, sb1)
        gsems = (g0, g1)
        ssems = (s0, s1)
        stsems = (t0, t1, t2, t3)

        def fb_pass(j, pcarry):
            fb = cid * (NFB // 2) + j

            def stage_start(q, tc):
                pltpu.async_copy(src_hbm.at[fb, sid, tc], esets[q], stsems[q])
                pltpu.async_copy(dst_hbm.at[sid, tc], dsets[q], stsems[q])
                pltpu.async_copy(attr_hbm.at[sid, tc], asets[q], stsems[q])

            def stage_wait(q, tc):
                pltpu.make_async_copy(src_hbm.at[fb, sid, tc], esets[q],
                                      stsems[q]).wait()
                pltpu.make_async_copy(dst_hbm.at[sid, tc], dsets[q],
                                      stsems[q]).wait()
                pltpu.make_async_copy(attr_hbm.at[sid, tc], asets[q],
                                      stsems[q]).wait()

            def gather_start(b, q):
                pltpu.async_copy(m_hbm.at[esets[q]], gbufs[b], gsems[b])

            def gather_wait(b, q):
                pltpu.make_async_copy(m_hbm.at[esets[q]], gbufs[b],
                                      gsems[b]).wait()

            def scatter_start(b, q):
                pltpu.async_copy(sbufs[b], slab.at[dsets[q]], ssems[b],
                                 add=True)

            def scatter_wait(b, q):
                pltpu.make_async_copy(sbufs[b], slab.at[dsets[q]],
                                      ssems[b]).wait()

            def mul(b, q):
                gbuf, sbuf, attr = gbufs[b], sbufs[b], asets[q]

                def edge4(e4i, carry):
                    for u4 in range(4):
                        e = e4i * 4 + u4
                        av = attr[e, :]
                        for c in range(D // 16):
                            sl = pl.ds(c * 16, 16)
                            sbuf[e, sl] = gbuf[e, sl] * av
                    return carry

                lax.fori_loop(0, ECHUNK // 4, edge4, 0)

            def zrow(r, carry):
                for c in range(D // 16):
                    sbufs[0][r, pl.ds(c * 16, 16)] = jnp.zeros((16,),
                                                               jnp.float32)
                return carry

            lax.fori_loop(0, ECHUNK, zrow, 0)
            for r0 in range(0, STRIPE - ECHUNK + 1, ECHUNK):
                pltpu.sync_copy(sbufs[0],
                                slab.at[pl.ds(sid * STRIPE + r0, ECHUNK)])
            _rem = STRIPE % ECHUNK
            if _rem:
                pltpu.sync_copy(
                    sbufs[0].at[pl.ds(0, _rem)],
                    slab.at[pl.ds(sid * STRIPE + STRIPE - _rem, _rem)])
            plsc.subcore_barrier()

            for q in (0, 1):
                stage_start(q, q)
                stage_wait(q, q)
                gather_start(q, q)

            def body(t4, carry):
                for u in range(NSET):
                    b = u % 2
                    qc = (u + 2) % NSET
                    t = NSET * t4 + u
                    if u < 2:
                        @pl.when(t4 > 0)
                        def _():
                            scatter_wait(b, qc)
                    else:
                        scatter_wait(b, qc)
                    tc = jnp.minimum(t + 2, NCHUNK - 1)
                    stage_start(qc, tc)
                    gather_wait(b, u)
                    mul(b, u)
                    scatter_start(b, u)
                    stage_wait(qc, tc)
                    gather_start(b, qc)
                return carry

            lax.fori_loop(0, NCHUNK // NSET, body, 0)
            scatter_wait(0, 2)
            scatter_wait(1, 3)
            gather_wait(0, 0)
            gather_wait(1, 1)

            plsc.subcore_barrier()
            pltpu.sync_copy(slab.at[pl.ds(sid * STRIPE, STRIPE)],
                            agg_hbm.at[pl.ds(fb * NP + sid * STRIPE, STRIPE)])
            plsc.subcore_barrier()
            return pcarry

        lax.fori_loop(0, NFB // 2, fb_pass, 0)

    return _sc_agg


def _sc_agg_call(m2, srcp, dstp, attrp):
    return _sc_agg_fn()(m2, srcp, dstp, attrp)


def _gate_pack(w):
    wt = w.T.reshape(H, 3, H)
    wt = jnp.pad(wt, ((0, HP - H), (0, 0), (0, HP - H)))
    return wt.reshape(HP, 3 * HP)


def _bias_pack(b):
    bp = jnp.pad(b.reshape(3, H), ((0, 0), (0, HP - H))).reshape(3 * HP)
    return jnp.broadcast_to(bp, (8, 3 * HP))


def kernel(x, edge_index, edge_attr, prob, weight, w_ih, w_hh, b_ih, b_hh,
           lin_w, lin_b):
    f32 = jnp.float32
    xp = jnp.pad(x.astype(f32), ((0, NP - N), (0, 0)))
    h = jnp.pad(xp, ((0, 0), (0, HP - D)))
    w0r = jnp.pad(weight[0][:D, :], ((0, 0), (0, HP - H)))
    wnext = {
        1: jnp.pad(weight[1], ((0, HP - H), (0, HP - H))).astype(jnp.bfloat16),
        2: jnp.pad(weight[2], ((0, HP - H), (0, HP - H))).astype(jnp.bfloat16),
    }
    wih_t = _gate_pack(w_ih).astype(jnp.bfloat16)
    whh_t = _gate_pack(w_hh).astype(jnp.bfloat16)
    bi = _bias_pack(b_ih)
    bh = _bias_pack(b_hh)
    srcp = jnp.pad(edge_index[0].astype(jnp.int32), (0, EP - E))
    srcp = (srcp[None, :] + (jnp.arange(NFB, dtype=jnp.int32) * NP)[:, None]
            ).reshape(NFB, NTILES, NCHUNK, ECHUNK)
    dstp = jnp.pad(edge_index[1].astype(jnp.int32),
                   (0, EP - E)).reshape(NTILES, NCHUNK, ECHUNK)
    attrp = jnp.pad(edge_attr.astype(f32), (0, EP - E))
    attrp = jnp.broadcast_to(attrp[:, None],
                             (EP, 16)).reshape(NTILES, NCHUNK, ECHUNK, 16)
    linp = jnp.zeros((HP, D), f32).at[:H, 0].set(lin_w[0])

    m = _mm0(xp, w0r)
    for layer in (1, 2):
        gh = _gh(h, whh_t, bh)
        agg = _sc_agg_call(m.reshape(NFB * NP, D), srcp, dstp, attrp)
        h, m = _gru_next(agg.reshape(NFB, NP, D), h, gh, wih_t, bi,
                         wnext[layer])
    gh = _gh(h, whh_t, bh)
    agg = _sc_agg_call(m.reshape(NFB * NP, D), srcp, dstp, attrp)
    out = _gru_final(agg.reshape(NFB, NP, D), h, gh, wih_t, bi, linp)
    return out[:N, :1] + lin_b[0]
